# Initial kernel scaffold; baseline (speedup 1.0000x reference)
import jax
import jax.numpy as jnp
from jax.experimental import pallas as pl


def kernel(pos_u, pos_w, neg_w, emb):
    raise NotImplementedError("write your pallas kernel here")


# speedup vs baseline: 20.3546x; 20.3546x over previous
import functools

import jax
import jax.numpy as jnp
from jax import lax
from jax.experimental import pallas as pl
from jax.experimental.pallas import tpu as pltpu
from jax.experimental.pallas import tpu_sc as plsc

VOCAB = 1000000
D = 32
C = 20
NEG = 5
B = 16384

NC = 2
NS = 16
NW = NC * NS
BPW = B // NW
CB = 64
NCHUNK = BPW // CB
IPB = C + 1 + NEG
ROWS = CB * IPB
NSTREAM = ROWS // 128
ISTRIDE = 16


def _sc_body(idx_hbm, emb_hbm, out_hbm, idx_v, rows_v, acc_v, sem0, sem1):
    wid = lax.axis_index("s") * NC + lax.axis_index("c")
    sems = (sem0, sem1)

    def fire(g, slot):
        pltpu.sync_copy(idx_hbm.at[wid, pl.ds(g * ISTRIDE, ISTRIDE)],
                        idx_v.at[slot])
        handles = []
        for j in range(NSTREAM):
            handles.append(pltpu.async_copy(
                emb_hbm.at[idx_v.at[slot, j]],
                rows_v.at[slot, pl.ds(j * 128, 128)],
                sems[slot]))
        return handles

    def compute(slot, accs):
        def body(b, accs):
            ap, a0, a1, a2, a3, a4 = accs
            cb = b * C
            ulo = rows_v[slot, cb, 0:16]
            uhi = rows_v[slot, cb, 16:32]
            for c in range(1, C):
                ulo = ulo + rows_v[slot, cb + c, 0:16]
                uhi = uhi + rows_v[slot, cb + c, 16:32]

            def dot(r):
                return ulo * rows_v[slot, r, 0:16] + uhi * rows_v[slot, r, 16:32]

            ap = ap + dot(CB * C + b)
            nb = CB * (C + 1) + b * NEG
            a0 = a0 + dot(nb)
            a1 = a1 + dot(nb + 1)
            a2 = a2 + dot(nb + 2)
            a3 = a3 + dot(nb + 3)
            a4 = a4 + dot(nb + 4)
            return (ap, a0, a1, a2, a3, a4)

        return lax.fori_loop(0, CB, body, accs)

    zero = jnp.zeros((16,), jnp.float32)
    accs = (zero, zero, zero, zero, zero, zero)
    handles = fire(0, 0)
    for g in range(NCHUNK):
        slot = g & 1
        nxt = fire(g + 1, slot ^ 1) if g + 1 < NCHUNK else None
        for h in handles:
            h.wait()
        accs = compute(slot, accs)
        handles = nxt
    for i in range(6):
        acc_v[i, 0:16] = accs[i]
    pltpu.sync_copy(acc_v, out_hbm.at[wid])


@functools.cache
def _cbow_sc():
    return pl.kernel(
        _sc_body,
        out_type=jax.ShapeDtypeStruct((NW, 8, 128), jnp.float32),
        mesh=plsc.VectorSubcoreMesh(core_axis_name="c", subcore_axis_name="s",
                                    num_cores=NC, num_subcores=NS),
        scratch_types=[
            pltpu.VMEM((2, ISTRIDE, 128), jnp.int32),
            pltpu.VMEM((2, ROWS, D), jnp.float32),
            pltpu.VMEM((8, 128), jnp.float32),
            pltpu.SemaphoreType.DMA,
            pltpu.SemaphoreType.DMA,
        ],
        compiler_params=pltpu.CompilerParams(use_tc_tiling_on_sc=False),
    )


def kernel(pos_u, pos_w, neg_w, emb):
    pos_u = pos_u.astype(jnp.int32)
    pos_w = pos_w.astype(jnp.int32)
    neg_w = neg_w.astype(jnp.int32)
    ctx = pos_u.reshape(NW, NCHUNK, CB * C)
    pw = pos_w.reshape(NW, NCHUNK, CB)
    ng = neg_w.reshape(NW, NCHUNK, CB * NEG)
    pad = jnp.zeros((NW, NCHUNK, (ISTRIDE - NSTREAM) * 128), jnp.int32)
    allidx = jnp.concatenate([ctx, pw, ng, pad], axis=-1)
    allidx = allidx.reshape(NW, NCHUNK * ISTRIDE, 128)
    partials = _cbow_sc()(allidx, emb)
    s = jnp.sum(partials[:, 0:6, 0:16], axis=(0, 2))
    return -jax.nn.log_sigmoid(s[0]) - jnp.sum(jax.nn.log_sigmoid(-s[1:]))
